# initial kernel scaffold (unmeasured)
import jax
import jax.numpy as jnp
from jax import lax
from jax.experimental import pallas as pl
from jax.experimental.pallas import tpu as pltpu

N_DEV = 4


def _allreduce_body(h_ref, out_ref, send_buf, comm_ref, send_sems, recv_sems):
    d = lax.axis_index("i")
    left = lax.rem(d + N_DEV - 1, N_DEV)
    right = lax.rem(d + 1, N_DEV)
    m_chunk = h_ref.shape[0] // N_DEV

    barrier_sem = pltpu.get_barrier_semaphore()
    for nbr in (left, right):
        pl.semaphore_signal(
            barrier_sem, inc=1,
            device_id=(nbr,), device_id_type=pl.DeviceIdType.MESH,
        )
    pl.semaphore_wait(barrier_sem, 2)

    send_buf[...] = h_ref[pl.ds(d * m_chunk, m_chunk), :]
    for s in range(N_DEV - 1):
        rdma = pltpu.make_async_remote_copy(
            src_ref=send_buf,
            dst_ref=comm_ref.at[s],
            send_sem=send_sems.at[s],
            recv_sem=recv_sems.at[s],
            device_id=(right,),
            device_id_type=pl.DeviceIdType.MESH,
        )
        rdma.start()
        rdma.wait()
        c = lax.rem(d + 2 * N_DEV - s - 1, N_DEV)
        send_buf[...] = comm_ref[s] + h_ref[pl.ds(c * m_chunk, m_chunk), :]

    f = lax.rem(d + 1, N_DEV)
    out_ref[pl.ds(f * m_chunk, m_chunk), :] = send_buf[...]

    for h in range(N_DEV - 1):
        rdma = pltpu.make_async_remote_copy(
            src_ref=send_buf,
            dst_ref=comm_ref.at[h],
            send_sem=send_sems.at[N_DEV - 1 + h],
            recv_sem=recv_sems.at[N_DEV - 1 + h],
            device_id=(right,),
            device_id_type=pl.DeviceIdType.MESH,
        )
        rdma.start()
        rdma.wait()
        c = lax.rem(d + 2 * N_DEV - h, N_DEV)
        out_ref[pl.ds(c * m_chunk, m_chunk), :] = comm_ref[h]
        if h < N_DEV - 2:
            send_buf[...] = comm_ref[h]


def kernel(x, W1, W2):
    xb = x.astype(jnp.bfloat16)
    W1b = W1.astype(jnp.bfloat16)
    W2b = W2.astype(jnp.bfloat16)
    h_partial = jnp.dot(
        xb, W1b, preferred_element_type=jnp.float32
    ).astype(jnp.bfloat16)
    m, k = h_partial.shape
    m_chunk = m // N_DEV

    h_full = pl.pallas_call(
        _allreduce_body,
        out_shape=jax.ShapeDtypeStruct((m, k), jnp.bfloat16),
        in_specs=[pl.BlockSpec(memory_space=pltpu.VMEM)],
        out_specs=pl.BlockSpec(memory_space=pltpu.VMEM),
        scratch_shapes=[
            pltpu.VMEM((m_chunk, k), jnp.bfloat16),
            pltpu.VMEM((N_DEV - 1, m_chunk, k), jnp.bfloat16),
            pltpu.SemaphoreType.DMA((2 * (N_DEV - 1),)),
            pltpu.SemaphoreType.DMA((2 * (N_DEV - 1),)),
        ],
        compiler_params=pltpu.CompilerParams(collective_id=0),
    )(h_partial)

    return jnp.dot(h_full, W2b, preferred_element_type=jnp.float32)


# baseline (device time: 754262 ns/iter reference)
import jax
import jax.numpy as jnp
from jax import lax
from jax.experimental import pallas as pl
from jax.experimental.pallas import tpu as pltpu

N_DEV = 4


def _allreduce_body(
    h_hbm, out_hbm, send_buf, comm_ref, local_buf,
    send_sems, recv_sems, load_sem, store_sem,
):
    d = lax.axis_index("i")
    left = lax.rem(d + N_DEV - 1, N_DEV)
    right = lax.rem(d + 1, N_DEV)
    m_chunk = h_hbm.shape[0] // N_DEV

    def chunk(ref, c):
        return ref.at[pl.ds(c * m_chunk, m_chunk), :]

    barrier_sem = pltpu.get_barrier_semaphore()
    for nbr in (left, right):
        pl.semaphore_signal(
            barrier_sem, inc=1,
            device_id=(nbr,), device_id_type=pl.DeviceIdType.MESH,
        )
    pl.semaphore_wait(barrier_sem, 2)

    cp = pltpu.make_async_copy(chunk(h_hbm, d), send_buf, load_sem)
    cp.start()
    cp.wait()

    for s in range(N_DEV - 1):
        rdma = pltpu.make_async_remote_copy(
            src_ref=send_buf,
            dst_ref=comm_ref.at[s],
            send_sem=send_sems.at[s],
            recv_sem=recv_sems.at[s],
            device_id=(right,),
            device_id_type=pl.DeviceIdType.MESH,
        )
        rdma.start()
        c = lax.rem(d + 2 * N_DEV - s - 1, N_DEV)
        cp = pltpu.make_async_copy(chunk(h_hbm, c), local_buf, load_sem)
        cp.start()
        rdma.wait()
        cp.wait()
        send_buf[...] = comm_ref[s] + local_buf[...]

    f = lax.rem(d + 1, N_DEV)
    st = pltpu.make_async_copy(send_buf, chunk(out_hbm, f), store_sem)
    st.start()
    st.wait()

    for h in range(N_DEV - 1):
        rdma = pltpu.make_async_remote_copy(
            src_ref=send_buf,
            dst_ref=comm_ref.at[h],
            send_sem=send_sems.at[N_DEV - 1 + h],
            recv_sem=recv_sems.at[N_DEV - 1 + h],
            device_id=(right,),
            device_id_type=pl.DeviceIdType.MESH,
        )
        rdma.start()
        rdma.wait()
        c = lax.rem(d + 2 * N_DEV - h, N_DEV)
        st = pltpu.make_async_copy(comm_ref.at[h], chunk(out_hbm, c), store_sem)
        st.start()
        if h < N_DEV - 2:
            send_buf[...] = comm_ref[h]
        st.wait()


def kernel(x, W1, W2):
    xb = x.astype(jnp.bfloat16)
    W1b = W1.astype(jnp.bfloat16)
    W2b = W2.astype(jnp.bfloat16)
    h_partial = jnp.dot(
        xb, W1b, preferred_element_type=jnp.float32
    ).astype(jnp.bfloat16)
    m, k = h_partial.shape
    m_chunk = m // N_DEV

    h_full = pl.pallas_call(
        _allreduce_body,
        out_shape=jax.ShapeDtypeStruct((m, k), jnp.bfloat16),
        in_specs=[pl.BlockSpec(memory_space=pl.ANY)],
        out_specs=pl.BlockSpec(memory_space=pl.ANY),
        scratch_shapes=[
            pltpu.VMEM((m_chunk, k), jnp.bfloat16),
            pltpu.VMEM((N_DEV - 1, m_chunk, k), jnp.bfloat16),
            pltpu.VMEM((m_chunk, k), jnp.bfloat16),
            pltpu.SemaphoreType.DMA((2 * (N_DEV - 1),)),
            pltpu.SemaphoreType.DMA((2 * (N_DEV - 1),)),
            pltpu.SemaphoreType.DMA,
            pltpu.SemaphoreType.DMA,
        ],
        compiler_params=pltpu.CompilerParams(
            collective_id=0, vmem_limit_bytes=60 * 1024 * 1024
        ),
    )(h_partial)

    return jnp.dot(h_full, W2b, preferred_element_type=jnp.float32)


# device time: 484710 ns/iter; 1.5561x vs baseline; 1.5561x over previous
import jax
import jax.numpy as jnp
from jax import lax
from jax.experimental import pallas as pl
from jax.experimental.pallas import tpu as pltpu

N_DEV = 4
N_HOP = 2 * (N_DEV - 1)


def _allreduce_body(
    h_hbm, out_hbm,
    sbA, sbB, commA, commB, locA, locB,
    ssA, rsA, ssB, rsB, ldA, ldB, stA, stB,
):
    d = lax.axis_index("i")
    left = lax.rem(d + N_DEV - 1, N_DEV)
    right = lax.rem(d + 1, N_DEV)
    m_chunk = h_hbm.shape[0] // N_DEV
    half = h_hbm.shape[1] // 2

    def hchunk(ref, c, off):
        return ref.at[pl.ds(c * m_chunk, m_chunk), pl.ds(off, half)]

    barrier_sem = pltpu.get_barrier_semaphore()
    for nbr in (left, right):
        pl.semaphore_signal(
            barrier_sem, inc=1,
            device_id=(nbr,), device_id_type=pl.DeviceIdType.MESH,
        )
    pl.semaphore_wait(barrier_sem, 2)

    cpA = pltpu.make_async_copy(hchunk(h_hbm, d, 0), sbA, ldA)
    cpB = pltpu.make_async_copy(hchunk(h_hbm, d, half), sbB, ldB)
    cpA.start()
    cpB.start()
    cpA.wait()
    cpB.wait()

    for s in range(N_DEV - 1):
        rdmaA = pltpu.make_async_remote_copy(
            src_ref=sbA, dst_ref=commA.at[s],
            send_sem=ssA.at[s], recv_sem=rsA.at[s],
            device_id=(right,), device_id_type=pl.DeviceIdType.MESH,
        )
        rdmaB = pltpu.make_async_remote_copy(
            src_ref=sbB, dst_ref=commB.at[s],
            send_sem=ssB.at[s], recv_sem=rsB.at[s],
            device_id=(left,), device_id_type=pl.DeviceIdType.MESH,
        )
        rdmaA.start()
        rdmaB.start()
        cA = lax.rem(d + 2 * N_DEV - s - 1, N_DEV)
        cB = lax.rem(d + s + 1, N_DEV)
        cpA = pltpu.make_async_copy(hchunk(h_hbm, cA, 0), locA, ldA)
        cpB = pltpu.make_async_copy(hchunk(h_hbm, cB, half), locB, ldB)
        cpA.start()
        cpB.start()
        rdmaA.wait()
        cpA.wait()
        sbA[...] = commA[s] + locA[...]
        rdmaB.wait()
        cpB.wait()
        sbB[...] = commB[s] + locB[...]

    fA = lax.rem(d + 1, N_DEV)
    fB = lax.rem(d + N_DEV - 1, N_DEV)
    outA = pltpu.make_async_copy(sbA, hchunk(out_hbm, fA, 0), stA)
    outB = pltpu.make_async_copy(sbB, hchunk(out_hbm, fB, half), stB)
    outA.start()
    outB.start()
    outA.wait()
    outB.wait()

    for h in range(N_DEV - 1):
        rdmaA = pltpu.make_async_remote_copy(
            src_ref=sbA, dst_ref=commA.at[h],
            send_sem=ssA.at[N_DEV - 1 + h], recv_sem=rsA.at[N_DEV - 1 + h],
            device_id=(right,), device_id_type=pl.DeviceIdType.MESH,
        )
        rdmaB = pltpu.make_async_remote_copy(
            src_ref=sbB, dst_ref=commB.at[h],
            send_sem=ssB.at[N_DEV - 1 + h], recv_sem=rsB.at[N_DEV - 1 + h],
            device_id=(left,), device_id_type=pl.DeviceIdType.MESH,
        )
        rdmaA.start()
        rdmaB.start()
        rdmaA.wait()
        rdmaB.wait()
        cA = lax.rem(d + 2 * N_DEV - h, N_DEV)
        cB = lax.rem(d + h, N_DEV)
        outA = pltpu.make_async_copy(commA.at[h], hchunk(out_hbm, cA, 0), stA)
        outB = pltpu.make_async_copy(commB.at[h], hchunk(out_hbm, cB, half), stB)
        outA.start()
        outB.start()
        if h < N_DEV - 2:
            sbA[...] = commA[h]
            sbB[...] = commB[h]
        outA.wait()
        outB.wait()


def kernel(x, W1, W2):
    xb = x.astype(jnp.bfloat16)
    W1b = W1.astype(jnp.bfloat16)
    W2b = W2.astype(jnp.bfloat16)
    h_partial = jnp.dot(
        xb, W1b, preferred_element_type=jnp.float32
    ).astype(jnp.bfloat16)
    m, k = h_partial.shape
    m_chunk = m // N_DEV
    half = k // 2

    h_full = pl.pallas_call(
        _allreduce_body,
        out_shape=jax.ShapeDtypeStruct((m, k), jnp.bfloat16),
        in_specs=[pl.BlockSpec(memory_space=pl.ANY)],
        out_specs=pl.BlockSpec(memory_space=pl.ANY),
        scratch_shapes=[
            pltpu.VMEM((m_chunk, half), jnp.bfloat16),
            pltpu.VMEM((m_chunk, half), jnp.bfloat16),
            pltpu.VMEM((N_DEV - 1, m_chunk, half), jnp.bfloat16),
            pltpu.VMEM((N_DEV - 1, m_chunk, half), jnp.bfloat16),
            pltpu.VMEM((m_chunk, half), jnp.bfloat16),
            pltpu.VMEM((m_chunk, half), jnp.bfloat16),
            pltpu.SemaphoreType.DMA((N_HOP,)),
            pltpu.SemaphoreType.DMA((N_HOP,)),
            pltpu.SemaphoreType.DMA((N_HOP,)),
            pltpu.SemaphoreType.DMA((N_HOP,)),
            pltpu.SemaphoreType.DMA,
            pltpu.SemaphoreType.DMA,
            pltpu.SemaphoreType.DMA,
            pltpu.SemaphoreType.DMA,
        ],
        compiler_params=pltpu.CompilerParams(
            collective_id=0, vmem_limit_bytes=60 * 1024 * 1024
        ),
    )(h_partial)

    return jnp.dot(h_full, W2b, preferred_element_type=jnp.float32)
